# Initial kernel scaffold; baseline (speedup 1.0000x reference)
import jax
import jax.numpy as jnp
from jax.experimental import pallas as pl


def kernel(x, embed):
    raise NotImplementedError("write your pallas kernel here")


# speedup vs baseline: 1.1028x; 1.1028x over previous
import functools

import jax
import jax.numpy as jnp
from jax import lax
from jax.experimental import pallas as pl
from jax.experimental.pallas import tpu as pltpu
from jax.experimental.pallas import tpu_sc as plsc

_B, _N, _D, _K = 16, 1024, 256, 8192
_BN = _B * _N

_TM = 512
_TK = 2048


def _argmin_dist_kernel(x_ref, embed_ref, xsq_ref, esq_ref, ind_ref, bval_ref):
    xt = x_ref[...]
    xsq = xsq_ref[...]

    def body(t, carry):
        b_val, b_idx = carry
        ech = embed_ref[pl.ds(t * _TK, _TK), :]
        esq = esq_ref[:, pl.ds(t * _TK, _TK)]
        scores = lax.dot_general(
            xt, ech, (((1,), (1,)), ((), ())),
            preferred_element_type=jnp.float32)
        dist = -(xsq - 2.0 * scores + esq)
        rowmax = jnp.max(dist, axis=1, keepdims=True)
        ids = lax.broadcasted_iota(jnp.int32, dist.shape, 1)
        cand = jnp.where(dist == rowmax, ids, jnp.int32(2**30))
        carg = jnp.min(cand, axis=1) + t * _TK
        cmax = rowmax[:, 0]
        upd = cmax > b_val
        return jnp.where(upd, cmax, b_val), jnp.where(upd, carg, b_idx)

    init = (jnp.full((_TM,), -jnp.inf, jnp.float32),
            jnp.zeros((_TM,), jnp.int32))
    b_val, b_idx = lax.fori_loop(0, _K // _TK, body, init)
    ind_ref[...] = b_idx
    bval_ref[...] = b_val


def _argmin_dist(xf, embed, xsq, esq):
    grid = (_BN // _TM,)
    return pl.pallas_call(
        _argmin_dist_kernel,
        grid=grid,
        in_specs=[
            pl.BlockSpec((_TM, _D), lambda i: (i, 0)),
            pl.BlockSpec((_K, _D), lambda i: (0, 0)),
            pl.BlockSpec((_TM, 1), lambda i: (i, 0)),
            pl.BlockSpec((1, _K), lambda i: (0, 0)),
        ],
        out_specs=[
            pl.BlockSpec((_TM,), lambda i: (i,)),
            pl.BlockSpec((_TM,), lambda i: (i,)),
        ],
        out_shape=[
            jax.ShapeDtypeStruct((_BN,), jnp.int32),
            jax.ShapeDtypeStruct((_BN,), jnp.float32),
        ],
    )(xf, embed, xsq, esq)


def _sc_gather(embed, ind):
    info = plsc.get_sparse_core_info()
    nw = info.num_cores * info.num_subcores
    bpw = _BN // nw
    chunk = 128
    mesh = plsc.VectorSubcoreMesh(core_axis_name="c", subcore_axis_name="s")

    @functools.partial(
        pl.kernel,
        out_type=jax.ShapeDtypeStruct((_BN, _D), jnp.float32),
        mesh=mesh,
        scratch_types=[
            pltpu.VMEM((bpw,), jnp.int32),
            pltpu.VMEM((chunk, _D), jnp.float32),
            pltpu.SemaphoreType.DMA,
        ],
    )
    def k(embed_hbm, idx_hbm, out_hbm, idx_v, rows_v, sem):
        wid = lax.axis_index("s") * info.num_cores + lax.axis_index("c")
        base = wid * bpw
        pltpu.sync_copy(idx_hbm.at[pl.ds(base, bpw)], idx_v)
        for c in range(bpw // chunk):
            pltpu.async_copy(
                embed_hbm.at[idx_v.at[pl.ds(c * chunk, chunk)]],
                rows_v, sem).wait()
            pltpu.sync_copy(rows_v, out_hbm.at[pl.ds(base + c * chunk, chunk)])

    return k(embed, ind)


def kernel(x, embed):
    xf = x.reshape(_BN, _D)
    xsq = jnp.sum(xf * xf, axis=-1, keepdims=True)
    esq = jnp.sum(embed * embed, axis=-1).reshape(1, _K)
    ind, bval = _argmin_dist(xf, embed, xsq, esq)
    quantize = _sc_gather(embed, ind).reshape(_B, _N, _D)
    embed_ind = ind.reshape(_B, _N)
    commit_loss = -jnp.sum(bval) / (_B * _N * _D)
    return quantize, embed_ind, commit_loss
